# baseline (speedup 1.0000x reference)
import functools

import jax
import jax.numpy as jnp
from jax import lax
from jax.experimental import pallas as pl
from jax.experimental.pallas import tpu as pltpu
from jax.experimental.pallas import tpu_sc as plsc

N = 10000
L = 16
E = 320000
V = 100000
D = 128
B = 64

NC = 2
NS = 16
NW = NC * NS
CHUNK = 80

NHALF = N // NC
EMB_ACC_ROWS = 5120
EMB_PER_TILE = EMB_ACC_ROWS
EMB_PER_SC = NS * EMB_PER_TILE
EMB_ROWS_PER_TILE = EMB_ACC_ROWS // NS
EMB_NCHUNKS = EMB_PER_TILE // CHUNK

ACC_ROWS = 10240
ROWS_PER_TILE = ACC_ROWS // NS
EP1 = 327680
EDGE_PER_TILE = EP1 // NW
EDGE_REAL_PER_TILE = E // NW
EDGE_PAD_PER_TILE = EDGE_PER_TILE - EDGE_REAL_PER_TILE
EDGE_NCHUNKS = EDGE_PER_TILE // CHUNK

_mesh = plsc.VectorSubcoreMesh(core_axis_name="c", subcore_axis_name="s")


def _gather_scatter_loop(table_hbm, src_hbm, dst_hbm, acc_sh,
                         src_v, dst_v, rows_v, gsems, ssems, isems,
                         idx_base, n_chunks):
    nq = n_chunks // 8

    def _idx_load(slot, ci):
        off = idx_base + ci * CHUNK
        a = pltpu.make_async_copy(src_hbm.at[pl.ds(off, CHUNK)],
                                  src_v.at[slot], isems[slot])
        b = pltpu.make_async_copy(dst_hbm.at[pl.ds(off, CHUNK)],
                                  dst_v.at[slot], isems[slot])
        return a, b

    def _gather(r, slot):
        return pltpu.make_async_copy(table_hbm.at[src_v.at[slot]],
                                     rows_v.at[r], gsems[r])

    def _scatter(r, slot):
        return pltpu.make_async_copy(rows_v.at[r], acc_sh.at[dst_v.at[slot]],
                                     ssems[r])

    for slot in range(2):
        a, b = _idx_load(slot, slot)
        a.start(); b.start(); a.wait(); b.wait()
        _gather(slot, slot).start()
    for slot in (2, 3):
        a, b = _idx_load(slot, slot)
        a.start(); b.start()

    @pl.loop(0, nq)
    def _(t):
        for b8 in range(8):
            ci = 8 * t + b8
            r = b8 % 4
            u = b8
            _gather(r, u).wait()
            _scatter(r, u).start(add=True)

            if b8 < 4:
                a, b = _idx_load((b8 + 4) % 8, ci + 4)
                a.start(); b.start()
            else:
                @pl.when(t < nq - 1)
                def _():
                    a, b = _idx_load((b8 + 4) % 8, ci + 4)
                    a.start(); b.start()

            r2 = (b8 + 2) % 4
            u2 = (b8 + 2) % 8

            def _launch(first=(b8 < 2)):
                if not first:
                    _scatter(r2, u2).wait()
                a, b = _idx_load(u2, ci + 2)
                a.wait(); b.wait()
                _gather(r2, u2).start()

            if b8 < 2:
                @pl.when(t == 0)
                def _():
                    _launch(True)

                @pl.when(t > 0)
                def _():
                    _launch(False)
            elif b8 < 6:
                _launch(False)
            else:
                @pl.when(t < nq - 1)
                def _():
                    _launch(False)

    _scatter(0, 4).wait()
    _scatter(1, 5).wait()
    _scatter(2, 6).wait()
    _scatter(3, 7).wait()


def _emb_body(table_hbm, src_hbm, dst_hbm, zeros_hbm, out_hbm,
              src_v, dst_v, rows_v, acc_sh, *sems):
    c = lax.axis_index("c")
    s = lax.axis_index("s")

    pltpu.sync_copy(zeros_hbm.at[pl.ds(0, EMB_ROWS_PER_TILE)],
                    acc_sh.at[pl.ds(s * EMB_ROWS_PER_TILE, EMB_ROWS_PER_TILE)])
    plsc.subcore_barrier()

    idx_base = c * EMB_PER_SC + s * EMB_PER_TILE
    _gather_scatter_loop(table_hbm, src_hbm, dst_hbm, acc_sh,
                         src_v, dst_v, rows_v, sems[0:4], sems[4:8],
                         sems[8:16], idx_base, EMB_NCHUNKS)

    plsc.subcore_barrier()
    local = s * EMB_ROWS_PER_TILE
    last_rows = NHALF - (NS - 1) * EMB_ROWS_PER_TILE

    @pl.when(s < NS - 1)
    def _():
        pltpu.sync_copy(
            acc_sh.at[pl.ds(local, EMB_ROWS_PER_TILE)],
            out_hbm.at[pl.ds(c * NHALF + local, EMB_ROWS_PER_TILE)])

    @pl.when(s == NS - 1)
    def _():
        pltpu.sync_copy(
            acc_sh.at[pl.ds(local, last_rows)],
            out_hbm.at[pl.ds(c * NHALF + local, last_rows)])


_emb_kernel = pl.kernel(
    _emb_body,
    out_type=jax.ShapeDtypeStruct((N, D), jnp.float32),
    mesh=_mesh,
    scratch_types=[
        pltpu.VMEM((8, CHUNK), jnp.int32),
        pltpu.VMEM((8, CHUNK), jnp.int32),
        pltpu.VMEM((4, CHUNK, D), jnp.float32),
        pltpu.VMEM_SHARED((EMB_ACC_ROWS, D), jnp.float32),
    ] + [pltpu.SemaphoreType.DMA] * 16,
)


def _edge_body(table_hbm, src_hbm, dst_hbm, zeros_hbm, out_hbm,
               src_v, dst_v, rows_v, acc_sh, *sems):
    c = lax.axis_index("c")
    s = lax.axis_index("s")
    wid = c * NS + s

    pltpu.sync_copy(zeros_hbm,
                    acc_sh.at[pl.ds(s * ROWS_PER_TILE, ROWS_PER_TILE)])
    plsc.subcore_barrier()

    _gather_scatter_loop(table_hbm, src_hbm, dst_hbm, acc_sh,
                         src_v, dst_v, rows_v, sems[0:4], sems[4:8],
                         sems[8:16], wid * EDGE_PER_TILE, EDGE_NCHUNKS)

    plsc.subcore_barrier()
    pltpu.sync_copy(
        acc_sh.at[pl.ds(s * ROWS_PER_TILE, ROWS_PER_TILE)],
        out_hbm.at[pl.ds(c * ACC_ROWS + s * ROWS_PER_TILE, ROWS_PER_TILE)])


_edge_kernel = pl.kernel(
    _edge_body,
    out_type=jax.ShapeDtypeStruct((NC * ACC_ROWS, D), jnp.float32),
    mesh=_mesh,
    scratch_types=[
        pltpu.VMEM((8, CHUNK), jnp.int32),
        pltpu.VMEM((8, CHUNK), jnp.int32),
        pltpu.VMEM((4, CHUNK, D), jnp.float32),
        pltpu.VMEM_SHARED((ACC_ROWS, D), jnp.float32),
    ] + [pltpu.SemaphoreType.DMA] * 16,
)

_ROW_BLK = 2000
_GRID = N // _ROW_BLK


def _layer_body(h_ref, q0_ref, q1_ref, w_ref, b_ref, o_ref):
    z = h_ref[...] + q0_ref[0] + q1_ref[0]
    y = jnp.dot(z, w_ref[...], preferred_element_type=jnp.float32) + b_ref[...]
    o_ref[...] = jnp.maximum(y, 0.0)


def _tc_layer(h, q, w, b):
    return pl.pallas_call(
        _layer_body,
        grid=(_GRID,),
        in_specs=[
            pl.BlockSpec((_ROW_BLK, D), lambda i: (i, 0)),
            pl.BlockSpec((1, _ROW_BLK, D), lambda i: (0, i, 0)),
            pl.BlockSpec((1, _ROW_BLK, D), lambda i: (1, i, 0)),
            pl.BlockSpec((D, D), lambda i: (0, 0)),
            pl.BlockSpec((1, D), lambda i: (0, 0)),
        ],
        out_specs=pl.BlockSpec((_ROW_BLK, D), lambda i: (i, 0)),
        out_shape=jax.ShapeDtypeStruct((N, D), jnp.float32),
    )(h, q, q, w, b.reshape(1, D))


def _pool_body(h_ref, r0_ref, r1_ref, w_ref, b_ref, batch_ref, o_ref):
    z = h_ref[...] + r0_ref[0] + r1_ref[0]
    h2 = jnp.maximum(
        jnp.dot(z, w_ref[...], preferred_element_type=jnp.float32) + b_ref[...], 0.0)
    bvec = batch_ref[0, 0, :]
    onehot = (bvec[:, None] == lax.broadcasted_iota(jnp.int32, (_ROW_BLK, B), 1)
              ).astype(jnp.float32)
    contrib = lax.dot_general(onehot, h2, (((0,), (0,)), ((), ())),
                              preferred_element_type=jnp.float32)

    @pl.when(pl.program_id(0) == 0)
    def _():
        o_ref[...] = jnp.zeros_like(o_ref)

    o_ref[...] += contrib


def _tc_pool(h, r, w, b, batch3):
    return pl.pallas_call(
        _pool_body,
        grid=(_GRID,),
        in_specs=[
            pl.BlockSpec((_ROW_BLK, D), lambda i: (i, 0)),
            pl.BlockSpec((1, _ROW_BLK, D), lambda i: (0, i, 0)),
            pl.BlockSpec((1, _ROW_BLK, D), lambda i: (1, i, 0)),
            pl.BlockSpec((D, D), lambda i: (0, 0)),
            pl.BlockSpec((1, D), lambda i: (0, 0)),
            pl.BlockSpec((1, 1, _ROW_BLK), lambda i: (i, 0, 0)),
        ],
        out_specs=pl.BlockSpec((B, D), lambda i: (0, 0)),
        out_shape=jax.ShapeDtypeStruct((B, D), jnp.float32),
    )(h, r, r, w, b.reshape(1, D), batch3)


def kernel(x, edge_index, batch, emb_table, W0, b0, W1, b1):
    x = x.astype(jnp.int32)

    tok_per_tile = NHALF * L // NS
    tok_pad = EMB_PER_TILE - tok_per_tile
    dst_local = jnp.repeat(jnp.arange(NHALF, dtype=jnp.int32), L)
    pad_src0 = jnp.broadcast_to(jnp.arange(tok_pad, dtype=jnp.int32),
                                (NS, tok_pad))
    pad_dst0 = jnp.broadcast_to(
        NHALF + jnp.arange(tok_pad, dtype=jnp.int32), (NS, tok_pad))
    halves_src = []
    halves_dst = []
    for c in range(NC):
        xs = x[c * NHALF:(c + 1) * NHALF].reshape(NS, tok_per_tile)
        halves_src.append(
            jnp.concatenate([xs, pad_src0], axis=1).reshape(-1))
        halves_dst.append(
            jnp.concatenate([dst_local.reshape(NS, tok_per_tile), pad_dst0],
                            axis=1).reshape(-1))
    src0 = jnp.concatenate(halves_src)
    dst0 = jnp.concatenate(halves_dst)

    pad_src1 = jnp.broadcast_to(
        jnp.arange(EDGE_PAD_PER_TILE, dtype=jnp.int32), (NW, EDGE_PAD_PER_TILE))
    pad_dst1 = jnp.broadcast_to(
        N + jnp.arange(EDGE_PAD_PER_TILE, dtype=jnp.int32),
        (NW, EDGE_PAD_PER_TILE))
    src1 = jnp.concatenate(
        [edge_index[0].astype(jnp.int32).reshape(NW, EDGE_REAL_PER_TILE),
         pad_src1], axis=1).reshape(-1)
    dst1 = jnp.concatenate(
        [edge_index[1].astype(jnp.int32).reshape(NW, EDGE_REAL_PER_TILE),
         pad_dst1], axis=1).reshape(-1)

    zeros_blk = jnp.zeros((ROWS_PER_TILE, D), jnp.float32)
    batch3 = batch.astype(jnp.int32).reshape(_GRID, 1, _ROW_BLK)

    h0 = _emb_kernel(emb_table, src0, dst0, zeros_blk)
    q = _edge_kernel(h0, src1, dst1, zeros_blk).reshape(NC, ACC_ROWS, D)
    h1 = _tc_layer(h0, q, W0, b0)
    r = _edge_kernel(h1, src1, dst1, zeros_blk).reshape(NC, ACC_ROWS, D)
    return _tc_pool(h1, r, W1, b1, batch3)


# speedup vs baseline: 1.0639x; 1.0639x over previous
import jax
import jax.numpy as jnp
from jax import lax
from jax.experimental import pallas as pl
from jax.experimental.pallas import tpu as pltpu
from jax.experimental.pallas import tpu_sc as plsc

N = 10000
L = 16
E = 320000
V = 100000
D = 128
B = 64

NC = 2
NS = 16
NW = NC * NS
CHUNK = 128

NHALF = N // NC
EMB_ACC_ROWS = 5120
EMB_PER_TILE = EMB_ACC_ROWS
EMB_PER_SC = NS * EMB_PER_TILE
EMB_ROWS_PER_TILE = EMB_ACC_ROWS // NS
EMB_NCHUNKS = EMB_PER_TILE // CHUNK

ACC_ROWS = 10240
ROWS_PER_TILE = ACC_ROWS // NS
EP1 = 327680
EDGE_PER_TILE = EP1 // NW
EDGE_REAL_PER_TILE = E // NW
EDGE_PAD_PER_TILE = EDGE_PER_TILE - EDGE_REAL_PER_TILE
EDGE_NCHUNKS = EDGE_PER_TILE // CHUNK

_mesh = plsc.VectorSubcoreMesh(core_axis_name="c", subcore_axis_name="s")


def _gather_scatter_loop(table_hbm, src_hbm, dst_hbm, acc_sh,
                         src_v, dst_v, rows_v, gsems, isems,
                         idx_base, n_chunks):
    nq = n_chunks // 4

    def _idx_load(slot, ci):
        off = idx_base + ci * CHUNK
        a = pltpu.make_async_copy(src_hbm.at[pl.ds(off, CHUNK)],
                                  src_v.at[slot], isems[slot])
        b = pltpu.make_async_copy(dst_hbm.at[pl.ds(off, CHUNK)],
                                  dst_v.at[slot], isems[slot])
        return a, b

    def _gather(buf, slot):
        return pltpu.make_async_copy(table_hbm.at[src_v.at[slot]],
                                     rows_v.at[buf], gsems[buf])

    for slot in range(2):
        a, b = _idx_load(slot, slot)
        a.start(); b.start(); a.wait(); b.wait()
        _gather(slot, slot).start()
    for slot in (2, 3):
        a, b = _idx_load(slot, slot)
        a.start(); b.start()

    @pl.loop(0, nq)
    def _(t):
        for b4 in range(4):
            ci = 4 * t + b4
            buf = b4 % 2
            _gather(buf, b4).wait()
            pltpu.sync_copy(rows_v.at[buf], acc_sh.at[dst_v.at[b4]], add=True)

            @pl.when(t < nq - 1)
            def _():
                a, b = _idx_load(b4, ci + 4)
                a.start(); b.start()

            nslot = (b4 + 2) % 4
            if b4 < 2:
                a, b = _idx_load(nslot, ci + 2)
                a.wait(); b.wait()
                _gather(buf, nslot).start()
            else:
                @pl.when(t < nq - 1)
                def _():
                    a, b = _idx_load(nslot, ci + 2)
                    a.wait(); b.wait()
                    _gather(buf, nslot).start()


def _emb_body(table_hbm, src_hbm, dst_hbm, zeros_hbm, out_hbm,
              src_v, dst_v, rows_v, acc_sh, gs0, gs1, is0, is1, is2, is3):
    c = lax.axis_index("c")
    s = lax.axis_index("s")

    pltpu.sync_copy(zeros_hbm.at[pl.ds(0, EMB_ROWS_PER_TILE)],
                    acc_sh.at[pl.ds(s * EMB_ROWS_PER_TILE, EMB_ROWS_PER_TILE)])
    plsc.subcore_barrier()

    idx_base = c * EMB_PER_SC + s * EMB_PER_TILE
    _gather_scatter_loop(table_hbm, src_hbm, dst_hbm, acc_sh,
                         src_v, dst_v, rows_v, (gs0, gs1),
                         (is0, is1, is2, is3), idx_base, EMB_NCHUNKS)

    plsc.subcore_barrier()
    local = s * EMB_ROWS_PER_TILE
    last_rows = NHALF - (NS - 1) * EMB_ROWS_PER_TILE

    @pl.when(s < NS - 1)
    def _():
        pltpu.sync_copy(
            acc_sh.at[pl.ds(local, EMB_ROWS_PER_TILE)],
            out_hbm.at[pl.ds(c * NHALF + local, EMB_ROWS_PER_TILE)])

    @pl.when(s == NS - 1)
    def _():
        pltpu.sync_copy(
            acc_sh.at[pl.ds(local, last_rows)],
            out_hbm.at[pl.ds(c * NHALF + local, last_rows)])


_emb_kernel = pl.kernel(
    _emb_body,
    out_type=jax.ShapeDtypeStruct((N, D), jnp.float32),
    mesh=_mesh,
    scratch_types=[
        pltpu.VMEM((4, CHUNK), jnp.int32),
        pltpu.VMEM((4, CHUNK), jnp.int32),
        pltpu.VMEM((2, CHUNK, D), jnp.float32),
        pltpu.VMEM_SHARED((EMB_ACC_ROWS, D), jnp.float32),
        pltpu.SemaphoreType.DMA,
        pltpu.SemaphoreType.DMA,
        pltpu.SemaphoreType.DMA,
        pltpu.SemaphoreType.DMA,
        pltpu.SemaphoreType.DMA,
        pltpu.SemaphoreType.DMA,
    ],
)


def _edge_body(table_hbm, src_hbm, dst_hbm, zeros_hbm, out_hbm,
               src_v, dst_v, rows_v, acc_sh, gs0, gs1, is0, is1, is2, is3):
    c = lax.axis_index("c")
    s = lax.axis_index("s")
    wid = c * NS + s

    pltpu.sync_copy(zeros_hbm,
                    acc_sh.at[pl.ds(s * ROWS_PER_TILE, ROWS_PER_TILE)])
    plsc.subcore_barrier()

    _gather_scatter_loop(table_hbm, src_hbm, dst_hbm, acc_sh,
                         src_v, dst_v, rows_v, (gs0, gs1),
                         (is0, is1, is2, is3), wid * EDGE_PER_TILE,
                         EDGE_NCHUNKS)

    plsc.subcore_barrier()
    pltpu.sync_copy(
        acc_sh.at[pl.ds(s * ROWS_PER_TILE, ROWS_PER_TILE)],
        out_hbm.at[pl.ds(c * ACC_ROWS + s * ROWS_PER_TILE, ROWS_PER_TILE)])


_edge_kernel = pl.kernel(
    _edge_body,
    out_type=jax.ShapeDtypeStruct((NC * ACC_ROWS, D), jnp.float32),
    mesh=_mesh,
    scratch_types=[
        pltpu.VMEM((4, CHUNK), jnp.int32),
        pltpu.VMEM((4, CHUNK), jnp.int32),
        pltpu.VMEM((2, CHUNK, D), jnp.float32),
        pltpu.VMEM_SHARED((ACC_ROWS, D), jnp.float32),
        pltpu.SemaphoreType.DMA,
        pltpu.SemaphoreType.DMA,
        pltpu.SemaphoreType.DMA,
        pltpu.SemaphoreType.DMA,
        pltpu.SemaphoreType.DMA,
        pltpu.SemaphoreType.DMA,
    ],
)

_ROW_BLK = 2000
_GRID = N // _ROW_BLK


def _layer_body(h_ref, q0_ref, q1_ref, w_ref, b_ref, o_ref):
    z = h_ref[...] + q0_ref[0] + q1_ref[0]
    y = jnp.dot(z, w_ref[...], preferred_element_type=jnp.float32) + b_ref[...]
    o_ref[...] = jnp.maximum(y, 0.0)


def _tc_layer(h, q, w, b):
    return pl.pallas_call(
        _layer_body,
        grid=(_GRID,),
        in_specs=[
            pl.BlockSpec((_ROW_BLK, D), lambda i: (i, 0)),
            pl.BlockSpec((1, _ROW_BLK, D), lambda i: (0, i, 0)),
            pl.BlockSpec((1, _ROW_BLK, D), lambda i: (1, i, 0)),
            pl.BlockSpec((D, D), lambda i: (0, 0)),
            pl.BlockSpec((1, D), lambda i: (0, 0)),
        ],
        out_specs=pl.BlockSpec((_ROW_BLK, D), lambda i: (i, 0)),
        out_shape=jax.ShapeDtypeStruct((N, D), jnp.float32),
    )(h, q, q, w, b.reshape(1, D))


def _pool_body(h_ref, r0_ref, r1_ref, w_ref, b_ref, batch_ref, o_ref):
    z = h_ref[...] + r0_ref[0] + r1_ref[0]
    h2 = jnp.maximum(
        jnp.dot(z, w_ref[...], preferred_element_type=jnp.float32) + b_ref[...], 0.0)
    bvec = batch_ref[0, 0, :]
    onehot = (bvec[:, None] == lax.broadcasted_iota(jnp.int32, (_ROW_BLK, B), 1)
              ).astype(jnp.float32)
    contrib = lax.dot_general(onehot, h2, (((0,), (0,)), ((), ())),
                              preferred_element_type=jnp.float32)

    @pl.when(pl.program_id(0) == 0)
    def _():
        o_ref[...] = jnp.zeros_like(o_ref)

    o_ref[...] += contrib


def _tc_pool(h, r, w, b, batch3):
    return pl.pallas_call(
        _pool_body,
        grid=(_GRID,),
        in_specs=[
            pl.BlockSpec((_ROW_BLK, D), lambda i: (i, 0)),
            pl.BlockSpec((1, _ROW_BLK, D), lambda i: (0, i, 0)),
            pl.BlockSpec((1, _ROW_BLK, D), lambda i: (1, i, 0)),
            pl.BlockSpec((D, D), lambda i: (0, 0)),
            pl.BlockSpec((1, D), lambda i: (0, 0)),
            pl.BlockSpec((1, 1, _ROW_BLK), lambda i: (i, 0, 0)),
        ],
        out_specs=pl.BlockSpec((B, D), lambda i: (0, 0)),
        out_shape=jax.ShapeDtypeStruct((B, D), jnp.float32),
    )(h, r, r, w, b.reshape(1, D), batch3)


def kernel(x, edge_index, batch, emb_table, W0, b0, W1, b1):
    x = x.astype(jnp.int32)

    tok_per_tile = NHALF * L // NS
    tok_pad = EMB_PER_TILE - tok_per_tile
    dst_local = jnp.repeat(jnp.arange(NHALF, dtype=jnp.int32), L)
    pad_src0 = jnp.broadcast_to(jnp.arange(tok_pad, dtype=jnp.int32),
                                (NS, tok_pad))
    pad_dst0 = jnp.broadcast_to(
        NHALF + jnp.arange(tok_pad, dtype=jnp.int32), (NS, tok_pad))
    halves_src = []
    halves_dst = []
    for c in range(NC):
        xs = x[c * NHALF:(c + 1) * NHALF].reshape(NS, tok_per_tile)
        halves_src.append(
            jnp.concatenate([xs, pad_src0], axis=1).reshape(-1))
        halves_dst.append(
            jnp.concatenate([dst_local.reshape(NS, tok_per_tile), pad_dst0],
                            axis=1).reshape(-1))
    src0 = jnp.concatenate(halves_src)
    dst0 = jnp.concatenate(halves_dst)

    pad_src1 = jnp.broadcast_to(
        jnp.arange(EDGE_PAD_PER_TILE, dtype=jnp.int32), (NW, EDGE_PAD_PER_TILE))
    pad_dst1 = jnp.broadcast_to(
        N + jnp.arange(EDGE_PAD_PER_TILE, dtype=jnp.int32),
        (NW, EDGE_PAD_PER_TILE))
    src1 = jnp.concatenate(
        [edge_index[0].astype(jnp.int32).reshape(NW, EDGE_REAL_PER_TILE),
         pad_src1], axis=1).reshape(-1)
    dst1 = jnp.concatenate(
        [edge_index[1].astype(jnp.int32).reshape(NW, EDGE_REAL_PER_TILE),
         pad_dst1], axis=1).reshape(-1)

    zeros_blk = jnp.zeros((ROWS_PER_TILE, D), jnp.float32)
    batch3 = batch.astype(jnp.int32).reshape(_GRID, 1, _ROW_BLK)

    h0 = _emb_kernel(emb_table, src0, dst0, zeros_blk)
    q = _edge_kernel(h0, src1, dst1, zeros_blk).reshape(NC, ACC_ROWS, D)
    h1 = _tc_layer(h0, q, W0, b0)
    r = _edge_kernel(h1, src1, dst1, zeros_blk).reshape(NC, ACC_ROWS, D)
    return _tc_pool(h1, r, W1, b1, batch3)
